# baseline (speedup 1.0000x reference)
import jax
import jax.numpy as jnp
from jax.experimental import pallas as pl
from jax.experimental.pallas import tpu as pltpu

N_ = 64
P_ = 64
CSP = 128
CDN = 64
CO = 32
M_ = 32
BBLK = 8
QUAD = 4
DLANES = P_ * CDN

_PREC = jax.lax.Precision.DEFAULT


def _dot(a, b):
    return jnp.dot(a, b, precision=_PREC, preferred_element_type=jnp.float32)


def _lrelu(x):
    return jnp.where(x > 0, x, 0.2 * x)


TBLK = 8


def _tr_body(src_ref, dst_ref):
    for b in range(TBLK):
        t = jnp.transpose(src_ref[b])
        t5 = t.reshape(N_, P_ // 4, 4, CDN)
        b0 = t5[:, :, 0:2, :].reshape(N_, P_ // 2, CDN)
        b1 = t5[:, :, 2:4, :].reshape(N_, P_ // 2, CDN)
        dst_ref[b] = jnp.concatenate([b0, b1], axis=1)


def _body(xt_ref, sf_ref, coor_ref, wsp_ref, wdn_ref, wc_ref,
          bsp_ref, ssp_ref, besp_ref,
          bdn_ref, sdn_ref, bedn_ref,
          bds_ref, sds_ref, beds_ref,
          spo_ref, outo_ref, coors_ref, sfps_scr):
    B = BBLK
    coor = coor_ref[...]
    lane_n = jax.lax.broadcasted_iota(jnp.int32, (B, N_), 1)

    def fps_step(t, carry):
        dists, far = carry
        onehot = (lane_n == far).astype(jnp.float32)
        sfps_scr[:, pl.ds(t, 1), :] = onehot[:, None, :]
        centroid = jnp.sum(coor * onehot[:, :, None], axis=1)
        coors_ref[:, pl.ds(t, 1), :] = centroid[:, None, :]
        d = jnp.sum((coor - centroid[:, None, :]) ** 2, axis=2)
        dists = jnp.minimum(dists, d)
        mx = jnp.max(dists, axis=1, keepdims=True)
        far = jnp.min(jnp.where(dists == mx, lane_n, N_), axis=1, keepdims=True)
        return dists, far

    carry0 = (jnp.full((B, N_), 1e10, jnp.float32),
              jnp.zeros((B, 1), jnp.int32))
    jax.lax.fori_loop(0, M_, fps_step, carry0)
    sfps = sfps_scr[...]
    centers = coors_ref[...]

    dc = jnp.sum((centers[:, :, None, :] - coor[:, None, :, :]) ** 2, axis=3)
    lane3 = jax.lax.broadcasted_iota(jnp.int32, (B, M_, N_), 2)
    mn0 = jnp.min(dc, axis=2, keepdims=True)
    i0 = jnp.min(jnp.where(dc == mn0, lane3, N_), axis=2, keepdims=True)
    s0 = lane3 == i0
    dc1 = jnp.where(s0, jnp.float32(jnp.inf), dc)
    mn1 = jnp.min(dc1, axis=2, keepdims=True)
    i1 = jnp.min(jnp.where(dc1 == mn1, lane3, N_), axis=2, keepdims=True)
    s1 = lane3 == i1
    sd0 = s0.astype(jnp.float32) - sfps
    sd1 = s1.astype(jnp.float32) - sfps

    wsp = wsp_ref[...]
    wdn = wdn_ref[...]
    wc = wc_ref[...]
    bsp = bsp_ref[...]; ssp = ssp_ref[...]; besp = besp_ref[...]
    bdn = bdn_ref[...]; sdn = sdn_ref[...]; bedn = bedn_ref[...]
    bds = bds_ref[...]; sds = sds_ref[...]; beds = beds_ref[...]
    Q = P_ // 2
    R96 = 3 * M_

    def process_quad(q):
        bs0 = q * QUAD
        rows = []
        for i in range(QUAD):
            b = bs0 + i
            cat = jnp.concatenate([sd0[b], sd1[b], sfps[b]], axis=0)
            pieces = []
            if i:
                pieces.append(jnp.zeros((R96, N_ * i), jnp.float32))
            pieces.append(cat)
            if QUAD - 1 - i:
                pieces.append(jnp.zeros((R96, N_ * (QUAD - 1 - i)), jnp.float32))
            rows.append(jnp.concatenate(pieces, axis=1))
        sbig = jnp.concatenate(rows, axis=0)

        xbig = xt_ref[pl.ds(bs0, QUAD)].reshape(QUAD * N_, DLANES)
        gbig = _dot(sbig, xbig)
        sfbig = sf_ref[pl.ds(bs0, QUAD)].reshape(QUAD * N_, CSP)
        gsbig = _dot(sbig, sfbig)

        ysp = []
        for i in range(QUAD):
            g = gsbig[i * R96:(i + 1) * R96]
            ysp.append(jnp.concatenate([g[0:M_], g[2 * M_:]], axis=1))
            ysp.append(jnp.concatenate([g[M_:2 * M_], g[2 * M_:]], axis=1))
        spv = _dot(jnp.concatenate(ysp, axis=0), wsp)
        for i in range(QUAD):
            sp0 = spv[2 * i * M_:(2 * i + 1) * M_]
            sp1 = spv[(2 * i + 1) * M_:(2 * i + 2) * M_]
            spm = (jnp.maximum(sp0, sp1) + bsp) * ssp + besp
            spo_ref[bs0 + i] = _lrelu(spm)

        ys = []
        for i in range(QUAD):
            g = gbig[i * R96:(i + 1) * R96]
            ys.append(g.reshape(3 * M_ * Q, 2 * CDN))
        ov = _dot(jnp.concatenate(ys, axis=0), wdn)

        taps = []
        for i in range(QUAD):
            o = ov[i * 3 * M_ * Q:(i + 1) * 3 * M_ * Q]
            o0 = o[0:M_ * Q]
            o1 = o[M_ * Q:2 * M_ * Q]
            oc = o[2 * M_ * Q:]
            ddf = _lrelu((jnp.maximum(o0, o1) + bdn) * sdn + bedn)
            ddc = _lrelu((oc + bdn) * sdn + bedn)
            ddf3 = ddf.reshape(M_, Q, CDN)
            ddc3 = ddc.reshape(M_, Q, CDN)
            even = jnp.concatenate([ddf3[:, :Q // 2, :], ddc3[:, :Q // 2, :]], axis=1)
            odd = jnp.concatenate([ddf3[:, Q // 2:, :], ddc3[:, Q // 2:, :]], axis=1)
            odd_sh = jnp.concatenate(
                [jnp.zeros((M_, 1, CDN), jnp.float32), odd[:, :Q - 1, :]], axis=1)
            taps.append(jnp.concatenate([even.reshape(M_ * Q, CDN),
                                         odd.reshape(M_ * Q, CDN),
                                         odd_sh.reshape(M_ * Q, CDN)], axis=1))
        cv = _dot(jnp.concatenate(taps, axis=0), wc)
        for i in range(QUAD):
            co = cv[i * M_ * Q:(i + 1) * M_ * Q]
            co = _lrelu((co + bds) * sds + beds)
            outo_ref[bs0 + i] = co.reshape(M_, Q, CDN)

    for q in range(BBLK // QUAD):
        process_quad(q)


def kernel(sparse_fea, dense_fea, stk_coor, n_stk_center,
           W_sp, b_sp, g_sp, be_sp,
           W_dn, b_dn, g_dn, be_dn,
           W_ds, b_ds, g_ds, be_ds):
    del n_stk_center
    bs = sparse_fea.shape[0]
    blk_t = lambda shape: pl.BlockSpec(shape, lambda i: (i,) + (0,) * (len(shape) - 1))
    xt4 = pl.pallas_call(
        _tr_body,
        grid=(bs // TBLK,),
        in_specs=[blk_t((TBLK, CDN, N_ * P_))],
        out_specs=blk_t((TBLK, N_, P_, CDN)),
        out_shape=jax.ShapeDtypeStruct((bs, N_, P_, CDN), jnp.float32),
    )(dense_fea.reshape(bs, CDN, N_ * P_))
    xt = xt4.reshape(bs, N_, DLANES)
    sf_t = jnp.swapaxes(sparse_fea, 1, 2)

    wsp_t = W_sp.T
    wdn_t = W_dn.T
    wct = jnp.transpose(W_ds[:, :, 0, :], (2, 1, 0))
    wc = jnp.concatenate([wct[1], wct[2], wct[0]], axis=0)
    inv = jnp.float32(1.0) / jnp.sqrt(jnp.float32(1.0 + 1e-5))
    row = lambda v: v.reshape(1, v.shape[0])

    grid = (bs // BBLK,)
    blk = lambda shape: pl.BlockSpec(shape, lambda i: (i,) + (0,) * (len(shape) - 1))
    full = lambda shape: pl.BlockSpec(shape, lambda i: (0,) * len(shape))

    sp_pre, out_pre, coor_s = pl.pallas_call(
        _body,
        grid=grid,
        in_specs=[blk((BBLK, N_, DLANES)), blk((BBLK, N_, CSP)),
                  blk((BBLK, N_, CO)),
                  full((2 * CSP, CSP)), full((2 * CDN, CDN)), full((3 * CDN, CDN)),
                  full((1, CSP)), full((1, CSP)), full((1, CSP)),
                  full((1, CDN)), full((1, CDN)), full((1, CDN)),
                  full((1, CDN)), full((1, CDN)), full((1, CDN))],
        out_specs=[blk((BBLK, M_, CSP)),
                   blk((BBLK, M_, P_ // 2, CDN)),
                   blk((BBLK, M_, CO))],
        out_shape=[jax.ShapeDtypeStruct((bs, M_, CSP), jnp.float32),
                   jax.ShapeDtypeStruct((bs, M_, P_ // 2, CDN), jnp.float32),
                   jax.ShapeDtypeStruct((bs, M_, CO), jnp.float32)],
        scratch_shapes=[pltpu.VMEM((BBLK, M_, N_), jnp.float32)],
    )(xt, sf_t, stk_coor, wsp_t, wdn_t, wc,
      row(b_sp), row(g_sp * inv), row(be_sp),
      row(b_dn), row(g_dn * inv), row(be_dn),
      row(b_ds), row(g_ds * inv), row(be_ds))

    sparse_out = jnp.transpose(sp_pre, (0, 2, 1))
    out = jnp.transpose(out_pre, (0, 3, 1, 2))
    return (sparse_out, out, coor_s)


# speedup vs baseline: 1.3088x; 1.3088x over previous
import jax
import jax.numpy as jnp
from jax.experimental import pallas as pl
from jax.experimental.pallas import tpu as pltpu

N_ = 64
P_ = 64
CSP = 128
CDN = 64
CO = 32
M_ = 32
BBLK = 8
QUAD = 4
DLANES = P_ * CDN

_PREC = jax.lax.Precision.DEFAULT


def _dot(a, b):
    return jnp.dot(a.astype(jnp.bfloat16), b.astype(jnp.bfloat16),
                   precision=_PREC, preferred_element_type=jnp.float32)


def _lrelu(x):
    return jnp.where(x > 0, x, 0.2 * x)


def _body(xt_ref, sf_ref, coor_ref, wsp_ref, wdn_ref, wc_ref,
          bsp_ref, ssp_ref, besp_ref,
          bdn_ref, sdn_ref, bedn_ref,
          bds_ref, sds_ref, beds_ref,
          spo_ref, outo_ref, coors_ref, sfps_scr):
    B = BBLK
    coor = coor_ref[...]
    lane_n = jax.lax.broadcasted_iota(jnp.int32, (B, N_), 1)

    def fps_step(t, carry):
        dists, far = carry
        onehot = (lane_n == far).astype(jnp.float32)
        sfps_scr[:, pl.ds(t, 1), :] = onehot[:, None, :]
        centroid = jnp.sum(coor * onehot[:, :, None], axis=1)
        coors_ref[:, pl.ds(t, 1), :] = centroid[:, None, :]
        d = jnp.sum((coor - centroid[:, None, :]) ** 2, axis=2)
        dists = jnp.minimum(dists, d)
        mx = jnp.max(dists, axis=1, keepdims=True)
        far = jnp.min(jnp.where(dists == mx, lane_n, N_), axis=1, keepdims=True)
        return dists, far

    carry0 = (jnp.full((B, N_), 1e10, jnp.float32),
              jnp.zeros((B, 1), jnp.int32))
    jax.lax.fori_loop(0, M_, fps_step, carry0)
    sfps = sfps_scr[...]
    centers = coors_ref[...]

    dc = jnp.sum((centers[:, :, None, :] - coor[:, None, :, :]) ** 2, axis=3)
    lane3 = jax.lax.broadcasted_iota(jnp.int32, (B, M_, N_), 2)
    mn0 = jnp.min(dc, axis=2, keepdims=True)
    i0 = jnp.min(jnp.where(dc == mn0, lane3, N_), axis=2, keepdims=True)
    s0 = lane3 == i0
    dc1 = jnp.where(s0, jnp.float32(jnp.inf), dc)
    mn1 = jnp.min(dc1, axis=2, keepdims=True)
    i1 = jnp.min(jnp.where(dc1 == mn1, lane3, N_), axis=2, keepdims=True)
    s1 = lane3 == i1
    sd0 = s0.astype(jnp.float32) - sfps
    sd1 = s1.astype(jnp.float32) - sfps

    wsp = wsp_ref[...]
    wdn = wdn_ref[...]
    wc = wc_ref[...]
    bsp = bsp_ref[...]; ssp = ssp_ref[...]; besp = besp_ref[...]
    bdn = bdn_ref[...]; sdn = sdn_ref[...]; bedn = bedn_ref[...]
    bds = bds_ref[...]; sds = sds_ref[...]; beds = beds_ref[...]
    Q = P_ // 2
    R96 = 3 * M_

    def process_quad(q):
        bs0 = q * QUAD
        rows = []
        for i in range(QUAD):
            b = bs0 + i
            cat = jnp.concatenate([sd0[b], sd1[b], sfps[b]], axis=0)
            pieces = []
            if i:
                pieces.append(jnp.zeros((R96, N_ * i), jnp.float32))
            pieces.append(cat)
            if QUAD - 1 - i:
                pieces.append(jnp.zeros((R96, N_ * (QUAD - 1 - i)), jnp.float32))
            rows.append(jnp.concatenate(pieces, axis=1))
        sbig = jnp.concatenate(rows, axis=0)

        xbig = xt_ref[pl.ds(bs0, QUAD)].reshape(QUAD * N_, DLANES)
        gbig = _dot(sbig, xbig)
        sfbig = sf_ref[pl.ds(bs0, QUAD)].reshape(QUAD * N_, CSP)
        gsbig = _dot(sbig, sfbig)

        ysp = []
        for i in range(QUAD):
            g = gsbig[i * R96:(i + 1) * R96]
            ysp.append(jnp.concatenate([g[0:M_], g[2 * M_:]], axis=1))
            ysp.append(jnp.concatenate([g[M_:2 * M_], g[2 * M_:]], axis=1))
        spv = _dot(jnp.concatenate(ysp, axis=0), wsp)
        for i in range(QUAD):
            sp0 = spv[2 * i * M_:(2 * i + 1) * M_]
            sp1 = spv[(2 * i + 1) * M_:(2 * i + 2) * M_]
            spm = (jnp.maximum(sp0, sp1) + bsp) * ssp + besp
            spo_ref[bs0 + i] = _lrelu(spm)

        ys = []
        for i in range(QUAD):
            g = gbig[i * R96:(i + 1) * R96]
            ys.append(g.reshape(3 * M_ * Q, 2 * CDN))
        ov = _dot(jnp.concatenate(ys, axis=0), wdn)

        taps = []
        for i in range(QUAD):
            o = ov[i * 3 * M_ * Q:(i + 1) * 3 * M_ * Q]
            o0 = o[0:M_ * Q]
            o1 = o[M_ * Q:2 * M_ * Q]
            oc = o[2 * M_ * Q:]
            ddf = _lrelu((jnp.maximum(o0, o1) + bdn) * sdn + bedn)
            ddc = _lrelu((oc + bdn) * sdn + bedn)
            ddf3 = ddf.reshape(M_, Q, CDN)
            ddc3 = ddc.reshape(M_, Q, CDN)
            even = jnp.concatenate([ddf3[:, :Q // 2, :], ddc3[:, :Q // 2, :]], axis=1)
            odd = jnp.concatenate([ddf3[:, Q // 2:, :], ddc3[:, Q // 2:, :]], axis=1)
            odd_sh = jnp.concatenate(
                [jnp.zeros((M_, 1, CDN), jnp.float32), odd[:, :Q - 1, :]], axis=1)
            taps.append(jnp.concatenate([even.reshape(M_ * Q, CDN),
                                         odd.reshape(M_ * Q, CDN),
                                         odd_sh.reshape(M_ * Q, CDN)], axis=1))
        cv = _dot(jnp.concatenate(taps, axis=0), wc)
        for i in range(QUAD):
            co = cv[i * M_ * Q:(i + 1) * M_ * Q]
            co = _lrelu((co + bds) * sds + beds)
            outo_ref[bs0 + i] = co.reshape(M_, Q, CDN)

    for q in range(BBLK // QUAD):
        process_quad(q)


def kernel(sparse_fea, dense_fea, stk_coor, n_stk_center,
           W_sp, b_sp, g_sp, be_sp,
           W_dn, b_dn, g_dn, be_dn,
           W_ds, b_ds, g_ds, be_ds):
    del n_stk_center
    bs = sparse_fea.shape[0]
    a_order = list(range(0, P_ // 2, 2)) + list(range(1, P_ // 2, 2))
    p_order = jnp.array([p for a in a_order for p in (2 * a, 2 * a + 1)],
                        dtype=jnp.int32)
    xt = (jnp.transpose(dense_fea, (0, 2, 3, 1))[:, :, p_order, :]
          .reshape(bs, N_, DLANES))
    sf_t = jnp.swapaxes(sparse_fea, 1, 2)

    wsp_t = W_sp.T
    wdn_t = W_dn.T
    wct = jnp.transpose(W_ds[:, :, 0, :], (2, 1, 0))
    wc = jnp.concatenate([wct[1], wct[2], wct[0]], axis=0)
    inv = jnp.float32(1.0) / jnp.sqrt(jnp.float32(1.0 + 1e-5))
    row = lambda v: v.reshape(1, v.shape[0])

    grid = (bs // BBLK,)
    blk = lambda shape: pl.BlockSpec(shape, lambda i: (i,) + (0,) * (len(shape) - 1))
    full = lambda shape: pl.BlockSpec(shape, lambda i: (0,) * len(shape))

    sp_pre, out_pre, coor_s = pl.pallas_call(
        _body,
        grid=grid,
        in_specs=[blk((BBLK, N_, DLANES)), blk((BBLK, N_, CSP)),
                  blk((BBLK, N_, CO)),
                  full((2 * CSP, CSP)), full((2 * CDN, CDN)), full((3 * CDN, CDN)),
                  full((1, CSP)), full((1, CSP)), full((1, CSP)),
                  full((1, CDN)), full((1, CDN)), full((1, CDN)),
                  full((1, CDN)), full((1, CDN)), full((1, CDN))],
        out_specs=[blk((BBLK, M_, CSP)),
                   blk((BBLK, M_, P_ // 2, CDN)),
                   blk((BBLK, M_, CO))],
        out_shape=[jax.ShapeDtypeStruct((bs, M_, CSP), jnp.float32),
                   jax.ShapeDtypeStruct((bs, M_, P_ // 2, CDN), jnp.float32),
                   jax.ShapeDtypeStruct((bs, M_, CO), jnp.float32)],
        scratch_shapes=[pltpu.VMEM((BBLK, M_, N_), jnp.float32)],
    )(xt, sf_t, stk_coor, wsp_t, wdn_t, wc,
      row(b_sp), row(g_sp * inv), row(be_sp),
      row(b_dn), row(g_dn * inv), row(be_dn),
      row(b_ds), row(g_ds * inv), row(be_ds))

    sparse_out = jnp.transpose(sp_pre, (0, 2, 1))
    out = jnp.transpose(out_pre, (0, 3, 1, 2))
    return (sparse_out, out, coor_s)


# speedup vs baseline: 1.4551x; 1.1118x over previous
import jax
import jax.numpy as jnp
from jax.experimental import pallas as pl
from jax.experimental.pallas import tpu as pltpu

N_ = 64
P_ = 64
CSP = 128
CDN = 64
CO = 32
M_ = 32
BBLK = 8
QUAD = 4
DLANES = P_ * CDN

_PREC = jax.lax.Precision.DEFAULT


def _dot(a, b):
    return jnp.dot(a.astype(jnp.bfloat16), b.astype(jnp.bfloat16),
                   precision=_PREC, preferred_element_type=jnp.float32)


def _lrelu(x):
    return jnp.where(x > 0, x, 0.2 * x)


def _body(xt_ref, sf_ref, coor_ref, wsp_ref, wdn_ref, wc_ref,
          bsp_ref, ssp_ref, besp_ref,
          bdn_ref, sdn_ref, bedn_ref,
          bds_ref, sds_ref, beds_ref,
          spo_ref, outo_ref, coors_ref, sfps_scr):
    B = BBLK
    coor = coor_ref[...]
    lane_n = jax.lax.broadcasted_iota(jnp.int32, (B, N_), 1)

    def fps_step(t, carry):
        dists, far = carry
        onehot = (lane_n == far).astype(jnp.float32)
        sfps_scr[:, pl.ds(t, 1), :] = onehot[:, None, :]
        centroid = jnp.sum(coor * onehot[:, :, None], axis=1)
        coors_ref[:, pl.ds(t, 1), :] = centroid[:, None, :]
        d = jnp.sum((coor - centroid[:, None, :]) ** 2, axis=2)
        dists = jnp.minimum(dists, d)
        mx = jnp.max(dists, axis=1, keepdims=True)
        far = jnp.min(jnp.where(dists == mx, lane_n, N_), axis=1, keepdims=True)
        return dists, far

    carry0 = (jnp.full((B, N_), 1e10, jnp.float32),
              jnp.zeros((B, 1), jnp.int32))
    jax.lax.fori_loop(0, M_, fps_step, carry0)
    sfps = sfps_scr[...]
    centers = coors_ref[...]

    dc = jnp.sum((centers[:, :, None, :] - coor[:, None, :, :]) ** 2, axis=3)
    lane3 = jax.lax.broadcasted_iota(jnp.int32, (B, M_, N_), 2)
    mn0 = jnp.min(dc, axis=2, keepdims=True)
    i0 = jnp.min(jnp.where(dc == mn0, lane3, N_), axis=2, keepdims=True)
    s0 = lane3 == i0
    dc1 = jnp.where(s0, jnp.float32(jnp.inf), dc)
    mn1 = jnp.min(dc1, axis=2, keepdims=True)
    i1 = jnp.min(jnp.where(dc1 == mn1, lane3, N_), axis=2, keepdims=True)
    s1 = lane3 == i1
    sd0 = s0.astype(jnp.float32) - sfps
    sd1 = s1.astype(jnp.float32) - sfps

    wsp = wsp_ref[...]
    wdn = wdn_ref[...]
    wc = wc_ref[...]
    bsp = bsp_ref[...]; ssp = ssp_ref[...]; besp = besp_ref[...]
    bdn = bdn_ref[...]; sdn = sdn_ref[...]; bedn = bedn_ref[...]
    bds = bds_ref[...]; sds = sds_ref[...]; beds = beds_ref[...]
    Q = P_ // 2
    R96 = 3 * M_

    def process_quad(q):
        bs0 = q * QUAD
        rows = []
        for i in range(QUAD):
            b = bs0 + i
            cat = jnp.concatenate([sd0[b], sd1[b], sfps[b]], axis=0)
            pieces = []
            if i:
                pieces.append(jnp.zeros((R96, N_ * i), jnp.float32))
            pieces.append(cat)
            if QUAD - 1 - i:
                pieces.append(jnp.zeros((R96, N_ * (QUAD - 1 - i)), jnp.float32))
            rows.append(jnp.concatenate(pieces, axis=1))
        sbig = jnp.concatenate(rows, axis=0)

        xbig = xt_ref[pl.ds(bs0, QUAD)].reshape(QUAD * N_, DLANES)
        gbig = _dot(sbig, xbig)
        sfbig = sf_ref[pl.ds(bs0, QUAD)].reshape(QUAD * N_, CSP)
        gsbig = _dot(sbig, sfbig)

        ysp = []
        for i in range(QUAD):
            g = gsbig[i * R96:(i + 1) * R96]
            ysp.append(jnp.concatenate([g[0:M_], g[2 * M_:]], axis=1))
            ysp.append(jnp.concatenate([g[M_:2 * M_], g[2 * M_:]], axis=1))
        spv = _dot(jnp.concatenate(ysp, axis=0), wsp)
        for i in range(QUAD):
            sp0 = spv[2 * i * M_:(2 * i + 1) * M_]
            sp1 = spv[(2 * i + 1) * M_:(2 * i + 2) * M_]
            spm = (jnp.maximum(sp0, sp1) + bsp) * ssp + besp
            spo_ref[bs0 + i] = _lrelu(spm)

        ys = []
        for i in range(QUAD):
            g = gbig[i * R96:(i + 1) * R96]
            ys.append(g.reshape(3 * M_ * Q, 2 * CDN))
        ov = _dot(jnp.concatenate(ys, axis=0), wdn)

        taps = []
        for i in range(QUAD):
            o = ov[i * 3 * M_ * Q:(i + 1) * 3 * M_ * Q]
            o0 = o[0:M_ * Q]
            o1 = o[M_ * Q:2 * M_ * Q]
            oc = o[2 * M_ * Q:]
            ddf = _lrelu((jnp.maximum(o0, o1) + bdn) * sdn + bedn)
            ddc = _lrelu((oc + bdn) * sdn + bedn)
            ddf3 = ddf.reshape(M_, Q, CDN)
            ddc3 = ddc.reshape(M_, Q, CDN)
            even = jnp.concatenate([ddf3[:, :Q // 2, :], ddc3[:, :Q // 2, :]], axis=1)
            odd = jnp.concatenate([ddf3[:, Q // 2:, :], ddc3[:, Q // 2:, :]], axis=1)
            odd_sh = jnp.concatenate(
                [jnp.zeros((M_, 1, CDN), jnp.float32), odd[:, :Q - 1, :]], axis=1)
            taps.append(jnp.concatenate([even.reshape(M_ * Q, CDN),
                                         odd.reshape(M_ * Q, CDN),
                                         odd_sh.reshape(M_ * Q, CDN)], axis=1))
        cv = _dot(jnp.concatenate(taps, axis=0), wc)
        for i in range(QUAD):
            co = cv[i * M_ * Q:(i + 1) * M_ * Q]
            co = _lrelu((co + bds) * sds + beds)
            outo_ref[bs0 + i] = co.reshape(M_, Q, CDN)

    for q in range(BBLK // QUAD):
        process_quad(q)


def kernel(sparse_fea, dense_fea, stk_coor, n_stk_center,
           W_sp, b_sp, g_sp, be_sp,
           W_dn, b_dn, g_dn, be_dn,
           W_ds, b_ds, g_ds, be_ds):
    del n_stk_center
    bs = sparse_fea.shape[0]
    a_order = list(range(0, P_ // 2, 2)) + list(range(1, P_ // 2, 2))
    p_order = jnp.array([p for a in a_order for p in (2 * a, 2 * a + 1)],
                        dtype=jnp.int32)
    xt = (jnp.transpose(dense_fea.astype(jnp.bfloat16), (0, 2, 3, 1))
          [:, :, p_order, :].reshape(bs, N_, DLANES))
    sf_t = jnp.swapaxes(sparse_fea, 1, 2)

    wsp_t = W_sp.T
    wdn_t = W_dn.T
    wct = jnp.transpose(W_ds[:, :, 0, :], (2, 1, 0))
    wc = jnp.concatenate([wct[1], wct[2], wct[0]], axis=0)
    inv = jnp.float32(1.0) / jnp.sqrt(jnp.float32(1.0 + 1e-5))
    row = lambda v: v.reshape(1, v.shape[0])

    grid = (bs // BBLK,)
    blk = lambda shape: pl.BlockSpec(shape, lambda i: (i,) + (0,) * (len(shape) - 1))
    full = lambda shape: pl.BlockSpec(shape, lambda i: (0,) * len(shape))

    sp_pre, out_pre, coor_s = pl.pallas_call(
        _body,
        grid=grid,
        in_specs=[blk((BBLK, N_, DLANES)), blk((BBLK, N_, CSP)),
                  blk((BBLK, N_, CO)),
                  full((2 * CSP, CSP)), full((2 * CDN, CDN)), full((3 * CDN, CDN)),
                  full((1, CSP)), full((1, CSP)), full((1, CSP)),
                  full((1, CDN)), full((1, CDN)), full((1, CDN)),
                  full((1, CDN)), full((1, CDN)), full((1, CDN))],
        out_specs=[blk((BBLK, M_, CSP)),
                   blk((BBLK, M_, P_ // 2, CDN)),
                   blk((BBLK, M_, CO))],
        out_shape=[jax.ShapeDtypeStruct((bs, M_, CSP), jnp.float32),
                   jax.ShapeDtypeStruct((bs, M_, P_ // 2, CDN), jnp.float32),
                   jax.ShapeDtypeStruct((bs, M_, CO), jnp.float32)],
        scratch_shapes=[pltpu.VMEM((BBLK, M_, N_), jnp.float32)],
    )(xt, sf_t, stk_coor, wsp_t, wdn_t, wc,
      row(b_sp), row(g_sp * inv), row(be_sp),
      row(b_dn), row(g_dn * inv), row(be_dn),
      row(b_ds), row(g_ds * inv), row(be_ds))

    sparse_out = jnp.transpose(sp_pre, (0, 2, 1))
    out = jnp.transpose(out_pre, (0, 3, 1, 2))
    return (sparse_out, out, coor_s)


# speedup vs baseline: 1.6771x; 1.1526x over previous
import jax
import jax.numpy as jnp
from jax.experimental import pallas as pl
from jax.experimental.pallas import tpu as pltpu

N_ = 64
P_ = 64
CSP = 128
CDN = 64
CO = 32
M_ = 32
BBLK = 16
QUAD = 4
DLANES = P_ * CDN

_PREC = jax.lax.Precision.DEFAULT


def _dot(a, b):
    return jnp.dot(a.astype(jnp.bfloat16), b.astype(jnp.bfloat16),
                   precision=_PREC, preferred_element_type=jnp.float32)


def _lrelu(x):
    return jnp.where(x > 0, x, 0.2 * x)


def _body(xt_ref, sf_ref, coor_ref, wsp_ref, wdn_ref, wc_ref,
          bsp_ref, ssp_ref, besp_ref,
          bdn_ref, sdn_ref, bedn_ref,
          bds_ref, sds_ref, beds_ref,
          spo_ref, outo_ref, coors_ref, sfps_scr):
    B = BBLK
    coor = coor_ref[...]
    lane_n = jax.lax.broadcasted_iota(jnp.int32, (B, N_), 1)

    def fps_step(t, carry):
        dists, far = carry
        onehot = (lane_n == far).astype(jnp.float32)
        sfps_scr[:, pl.ds(t, 1), :] = onehot[:, None, :]
        centroid = jnp.sum(coor * onehot[:, :, None], axis=1)
        coors_ref[:, pl.ds(t, 1), :] = centroid[:, None, :]
        d = jnp.sum((coor - centroid[:, None, :]) ** 2, axis=2)
        dists = jnp.minimum(dists, d)
        mx = jnp.max(dists, axis=1, keepdims=True)
        far = jnp.min(jnp.where(dists == mx, lane_n, N_), axis=1, keepdims=True)
        return dists, far

    carry0 = (jnp.full((B, N_), 1e10, jnp.float32),
              jnp.zeros((B, 1), jnp.int32))
    jax.lax.fori_loop(0, M_, fps_step, carry0)
    sfps = sfps_scr[...]
    centers = coors_ref[...]

    dc = jnp.sum((centers[:, :, None, :] - coor[:, None, :, :]) ** 2, axis=3)
    lane3 = jax.lax.broadcasted_iota(jnp.int32, (B, M_, N_), 2)
    mn0 = jnp.min(dc, axis=2, keepdims=True)
    i0 = jnp.min(jnp.where(dc == mn0, lane3, N_), axis=2, keepdims=True)
    s0 = lane3 == i0
    dc1 = jnp.where(s0, jnp.float32(jnp.inf), dc)
    mn1 = jnp.min(dc1, axis=2, keepdims=True)
    i1 = jnp.min(jnp.where(dc1 == mn1, lane3, N_), axis=2, keepdims=True)
    s1 = lane3 == i1
    sd0 = s0.astype(jnp.float32) - sfps
    sd1 = s1.astype(jnp.float32) - sfps

    wsp = wsp_ref[...]
    wdn = wdn_ref[...]
    wc = wc_ref[...]
    bsp = bsp_ref[...]; ssp = ssp_ref[...]; besp = besp_ref[...]
    bdn = bdn_ref[...]; sdn = sdn_ref[...]; bedn = bedn_ref[...]
    bds = bds_ref[...]; sds = sds_ref[...]; beds = beds_ref[...]
    Q = P_ // 2
    R96 = 3 * M_

    def process_quad(q):
        bs0 = q * QUAD
        rows = []
        for i in range(QUAD):
            b = bs0 + i
            cat = jnp.concatenate([sd0[b], sd1[b], sfps[b]], axis=0)
            pieces = []
            if i:
                pieces.append(jnp.zeros((R96, N_ * i), jnp.float32))
            pieces.append(cat)
            if QUAD - 1 - i:
                pieces.append(jnp.zeros((R96, N_ * (QUAD - 1 - i)), jnp.float32))
            rows.append(jnp.concatenate(pieces, axis=1))
        sbig = jnp.concatenate(rows, axis=0)

        xbig = xt_ref[pl.ds(bs0, QUAD)].reshape(QUAD * N_, DLANES)
        gbig = _dot(sbig, xbig)
        sfbig = sf_ref[pl.ds(bs0, QUAD)].reshape(QUAD * N_, CSP)
        gsbig = _dot(sbig, sfbig)

        ysp = []
        for i in range(QUAD):
            g = gsbig[i * R96:(i + 1) * R96]
            ysp.append(jnp.concatenate([g[0:M_], g[2 * M_:]], axis=1))
            ysp.append(jnp.concatenate([g[M_:2 * M_], g[2 * M_:]], axis=1))
        spv = _dot(jnp.concatenate(ysp, axis=0), wsp)
        for i in range(QUAD):
            sp0 = spv[2 * i * M_:(2 * i + 1) * M_]
            sp1 = spv[(2 * i + 1) * M_:(2 * i + 2) * M_]
            spm = (jnp.maximum(sp0, sp1) + bsp) * ssp + besp
            spo_ref[bs0 + i] = _lrelu(spm)

        ys = []
        for i in range(QUAD):
            g = gbig[i * R96:(i + 1) * R96]
            ys.append(g.reshape(3 * M_ * Q, 2 * CDN))
        ov = _dot(jnp.concatenate(ys, axis=0), wdn)

        taps = []
        for i in range(QUAD):
            o = ov[i * 3 * M_ * Q:(i + 1) * 3 * M_ * Q]
            o0 = o[0:M_ * Q]
            o1 = o[M_ * Q:2 * M_ * Q]
            oc = o[2 * M_ * Q:]
            ddf = _lrelu((jnp.maximum(o0, o1) + bdn) * sdn + bedn)
            ddc = _lrelu((oc + bdn) * sdn + bedn)
            ddf3 = ddf.reshape(M_, Q, CDN)
            ddc3 = ddc.reshape(M_, Q, CDN)
            even = jnp.concatenate([ddf3[:, :Q // 2, :], ddc3[:, :Q // 2, :]], axis=1)
            odd = jnp.concatenate([ddf3[:, Q // 2:, :], ddc3[:, Q // 2:, :]], axis=1)
            odd_sh = jnp.concatenate(
                [jnp.zeros((M_, 1, CDN), jnp.float32), odd[:, :Q - 1, :]], axis=1)
            taps.append(jnp.concatenate([even.reshape(M_ * Q, CDN),
                                         odd.reshape(M_ * Q, CDN),
                                         odd_sh.reshape(M_ * Q, CDN)], axis=1))
        cv = _dot(jnp.concatenate(taps, axis=0), wc)
        for i in range(QUAD):
            co = cv[i * M_ * Q:(i + 1) * M_ * Q]
            co = _lrelu((co + bds) * sds + beds)
            outo_ref[bs0 + i] = co.reshape(M_, Q, CDN)

    for q in range(BBLK // QUAD):
        process_quad(q)


def kernel(sparse_fea, dense_fea, stk_coor, n_stk_center,
           W_sp, b_sp, g_sp, be_sp,
           W_dn, b_dn, g_dn, be_dn,
           W_ds, b_ds, g_ds, be_ds):
    del n_stk_center
    bs = sparse_fea.shape[0]
    a_order = list(range(0, P_ // 2, 2)) + list(range(1, P_ // 2, 2))
    p_order = jnp.array([p for a in a_order for p in (2 * a, 2 * a + 1)],
                        dtype=jnp.int32)
    xt = (jnp.transpose(dense_fea.astype(jnp.bfloat16), (0, 2, 3, 1))
          [:, :, p_order, :].reshape(bs, N_, DLANES))
    sf_t = jnp.swapaxes(sparse_fea, 1, 2)

    wsp_t = W_sp.T
    wdn_t = W_dn.T
    wct = jnp.transpose(W_ds[:, :, 0, :], (2, 1, 0))
    wc = jnp.concatenate([wct[1], wct[2], wct[0]], axis=0)
    inv = jnp.float32(1.0) / jnp.sqrt(jnp.float32(1.0 + 1e-5))
    row = lambda v: v.reshape(1, v.shape[0])

    grid = (bs // BBLK,)
    blk = lambda shape: pl.BlockSpec(shape, lambda i: (i,) + (0,) * (len(shape) - 1))
    full = lambda shape: pl.BlockSpec(shape, lambda i: (0,) * len(shape))

    sp_pre, out_pre, coor_s = pl.pallas_call(
        _body,
        grid=grid,
        in_specs=[blk((BBLK, N_, DLANES)), blk((BBLK, N_, CSP)),
                  blk((BBLK, N_, CO)),
                  full((2 * CSP, CSP)), full((2 * CDN, CDN)), full((3 * CDN, CDN)),
                  full((1, CSP)), full((1, CSP)), full((1, CSP)),
                  full((1, CDN)), full((1, CDN)), full((1, CDN)),
                  full((1, CDN)), full((1, CDN)), full((1, CDN))],
        out_specs=[blk((BBLK, M_, CSP)),
                   blk((BBLK, M_, P_ // 2, CDN)),
                   blk((BBLK, M_, CO))],
        out_shape=[jax.ShapeDtypeStruct((bs, M_, CSP), jnp.float32),
                   jax.ShapeDtypeStruct((bs, M_, P_ // 2, CDN), jnp.float32),
                   jax.ShapeDtypeStruct((bs, M_, CO), jnp.float32)],
        scratch_shapes=[pltpu.VMEM((BBLK, M_, N_), jnp.float32)],
    )(xt, sf_t, stk_coor, wsp_t, wdn_t, wc,
      row(b_sp), row(g_sp * inv), row(be_sp),
      row(b_dn), row(g_dn * inv), row(be_dn),
      row(b_ds), row(g_ds * inv), row(be_ds))

    sparse_out = jnp.transpose(sp_pre, (0, 2, 1))
    out = jnp.transpose(out_pre, (0, 3, 1, 2))
    return (sparse_out, out, coor_s)
